# baseline (speedup 1.0000x reference)
import functools

import jax
import jax.numpy as jnp
from jax import lax
from jax.experimental import pallas as pl
from jax.experimental.pallas import tpu as pltpu
from jax.experimental.pallas import tpu_sc as plsc

N = 10000
E = 160000
D = 256
EDGE_D = 16
N_ATOM = 100
N_EDGE_T = 16
HID = 256
CW = 128
WID = D + CW

NC = 2
NS = 16
NW = NC * NS
L = 16

ECH = 128
NCHUNK = E // ECH
G_ITERS = -(-NCHUNK // NW)
S_ITERS = -(-NCHUNK // NS)

SC_N = N // NC
ACC_N = 5120
TRASH = 5100
ACC_R = ACC_N // 8

BN = 2000
GN = N // BN
BP = SC_N
GP = NC
BE = 3200
GE = E // BE


def _silu(x):
    return x * jax.nn.sigmoid(x)


def _node_pre_body(af_ref, c128_ref, atab_ref, w1ab_ref,
                   h_ref, a_ref, b_ref):
    ids = af_ref[0, 0, :]
    oh = (ids[:, None] == lax.broadcasted_iota(jnp.int32, (BN, N_ATOM), 1)
          ).astype(jnp.float32)
    h = jnp.dot(oh, atab_ref[...], preferred_element_type=jnp.float32)
    h_ref[...] = h
    c = c128_ref[...]
    a_ref[:, 0:D] = jnp.dot(h, w1ab_ref[0:D, :], preferred_element_type=jnp.float32)
    a_ref[:, D:WID] = c
    b_ref[:, 0:D] = jnp.dot(h, w1ab_ref[D:2 * D, :], preferred_element_type=jnp.float32)
    b_ref[:, D:WID] = -c


def _node_pre_t_body(etab_ref, w1d_ref, t_ref):
    t_ref[...] = jnp.dot(etab_ref[...], w1d_ref[...],
                         preferred_element_type=jnp.float32)


def _edge_body(ga_ref, gb_ref, et_ref, row_ref, t_ref, wr_ref, be1_ref,
               we2_ref, be2_ref, wx1_ref, bx1_ref, wx2_ref, etab_ref,
               m_ref, t16_ref, ea_ref, rc0_ref, rc1_ref):
    cd = ga_ref[:, D:WID] + gb_ref[:, D:WID]
    radial = jnp.sum(cd * cd, axis=1, keepdims=True)
    ids = et_ref[0, 0, :]
    oh = (ids[:, None] == lax.broadcasted_iota(jnp.int32, (BE, N_EDGE_T), 1)
          ).astype(jnp.float32)
    pre = (ga_ref[:, 0:D] + gb_ref[:, 0:D] + radial * wr_ref[...]
           + jnp.dot(oh, t_ref[...], preferred_element_type=jnp.float32)
           + be1_ref[...])
    m1 = _silu(pre)
    m = _silu(jnp.dot(m1, we2_ref[...], preferred_element_type=jnp.float32)
              + be2_ref[...])
    m_ref[...] = m
    m2 = _silu(jnp.dot(m, wx1_ref[...], preferred_element_type=jnp.float32)
               + bx1_ref[...])
    g = jnp.sum(m2 * wx2_ref[...], axis=1, keepdims=True)
    lane = lax.broadcasted_iota(jnp.int32, (BE, EDGE_D), 1)
    t16_ref[...] = jnp.where(lane == 3, 1.0, cd[:, 0:EDGE_D] * g)
    ea_ref[...] = jnp.dot(oh, etab_ref[...], preferred_element_type=jnp.float32)
    r = row_ref[0, 0, :][:, None]
    rc0_ref[...] = jnp.broadcast_to(jnp.where(r < SC_N, r, TRASH), (BE, L))
    rc1_ref[...] = jnp.broadcast_to(jnp.where(r >= SC_N, r - SC_N, TRASH),
                                    (BE, L))


def _node_post_body(h_ref, aggm_ref, pt_ref, c16p_ref,
                    wh1_ref, bh1_ref, wh2_ref, bh2_ref,
                    hout_ref, cout_ref):
    h = h_ref[...]
    u = _silu(jnp.dot(h, wh1_ref[0:D, :], preferred_element_type=jnp.float32)
              + jnp.dot(aggm_ref[...], wh1_ref[D:D + HID, :],
                        preferred_element_type=jnp.float32)
              + bh1_ref[...])
    hout_ref[...] = h + jnp.dot(u, wh2_ref[...],
                                preferred_element_type=jnp.float32) + bh2_ref[...]
    acp = jnp.sum(pt_ref[...], axis=(0, 1))
    rr = lax.broadcasted_iota(jnp.int32, (128, 128), 0)
    cc = lax.broadcasted_iota(jnp.int32, (128, 128), 1)
    msel = (((rr // L) == (cc // L)) & (rr % L == 3)).astype(jnp.float32)
    deg = jnp.dot(acp, msel, preferred_element_type=jnp.float32)
    cout_ref[0] = c16p_ref[0] + acp / jnp.maximum(deg, 1.0)


_sc_mesh = plsc.VectorSubcoreMesh(core_axis_name="c", subcore_axis_name="s")


@functools.partial(
    pl.kernel,
    out_type=(jax.ShapeDtypeStruct((E, WID), jnp.float32),
              jax.ShapeDtypeStruct((E, WID), jnp.float32)),
    scratch_types=[
        pltpu.VMEM((ECH,), jnp.int32),
        pltpu.VMEM((ECH,), jnp.int32),
        pltpu.VMEM((ECH, WID), jnp.float32),
        pltpu.VMEM((ECH, WID), jnp.float32),
        pltpu.SemaphoreType.DMA,
        pltpu.SemaphoreType.DMA,
    ],
    mesh=_sc_mesh,
)
def _sc_gather(a_hbm, b_hbm, row_hbm, col_hbm,
               ga_hbm, gb_hbm, idx1_v, idx2_v, gbuf, gbuf2, sem, sem2):
    c = lax.axis_index("c")
    s = lax.axis_index("s")
    w = s * NC + c

    def body(i, carry):
        ch = w + i * NW

        @pl.when(ch < NCHUNK)
        def _():
            base = ch * ECH
            pltpu.sync_copy(row_hbm.at[pl.ds(base, ECH)], idx1_v)
            pltpu.sync_copy(col_hbm.at[pl.ds(base, ECH)], idx2_v)
            cp1 = pltpu.async_copy(a_hbm.at[idx1_v], gbuf, sem)
            cp2 = pltpu.async_copy(b_hbm.at[idx2_v], gbuf2, sem2)
            cp1.wait()
            cp2.wait()
            pltpu.sync_copy(gbuf, ga_hbm.at[pl.ds(base, ECH), :])
            pltpu.sync_copy(gbuf2, gb_hbm.at[pl.ds(base, ECH), :])

        return carry

    lax.fori_loop(0, G_ITERS, body, 0)


@functools.partial(
    pl.kernel,
    out_type=(jax.ShapeDtypeStruct((NC, NS, SC_N // 8, 128), jnp.float32),
              jax.ShapeDtypeStruct((NC, NS, ACC_R, 128), jnp.float32)),
    scratch_types=[
        pltpu.VMEM((ACC_R, 128), jnp.float32),
        pltpu.VMEM((ECH // 8, 128), jnp.float32),
        pltpu.VMEM((ECH // 8, 128), jnp.int32),
    ],
    mesh=_sc_mesh,
    compiler_params=pltpu.CompilerParams(needs_layout_passes=False),
)
def _sc_scatter(m_hbm, t16_hbm, rc_hbm, aggm_hbm, partt_hbm, acc, vbuf, rbuf):
    c = lax.axis_index("c")
    s = lax.axis_index("s")
    iota = lax.broadcasted_iota(jnp.int32, (L,), 0)
    zero = jnp.zeros((L,), jnp.float32)

    def zr(r, carry):
        for k in range(128 // L):
            acc[r, pl.ds(k * L, L)] = zero
        return carry

    def scatter_rows(j):
        r_, lo = j >> 3, (j & 7) * L
        rb = rbuf[r_, pl.ds(lo, L)]
        rowv = lax.shift_right_logical(rb, 3)
        colv = lax.shift_left(jnp.bitwise_and(rb, 7), 4) + iota
        plsc.addupdate_scatter(acc, [rowv, colv], vbuf[r_, pl.ds(lo, L)])

    lax.fori_loop(0, ACC_R, zr, 0)

    def body(i, carry):
        base = i * (ECH // 8)
        pltpu.sync_copy(m_hbm.at[s, pl.ds(base, ECH // 8), :], vbuf)
        pltpu.sync_copy(rc_hbm.at[c, pl.ds(base, ECH // 8), :], rbuf)
        for j in range(ECH):
            scatter_rows(j)
        return carry

    lax.fori_loop(0, NCHUNK, body, 0)
    pltpu.sync_copy(acc.at[pl.ds(0, SC_N // 8), :], aggm_hbm.at[c, s])

    lax.fori_loop(0, ACC_R, zr, 0)

    def body2(i, carry):
        ch = s + i * NS

        @pl.when(ch < NCHUNK)
        def _():
            base = ch * (ECH // 8)
            pltpu.sync_copy(t16_hbm.at[pl.ds(base, ECH // 8), :], vbuf)
            pltpu.sync_copy(rc_hbm.at[c, pl.ds(base, ECH // 8), :], rbuf)
            for j in range(ECH):
                scatter_rows(j)

        return carry

    lax.fori_loop(0, S_ITERS, body2, 0)
    pltpu.sync_copy(acc, partt_hbm.at[c, s])


def _full(x):
    return pl.BlockSpec(x, lambda *_: tuple(0 for _ in x))


def kernel(atom_feats, coord, edge_index, edge_type_ids, atom_table,
           edge_table, W_e1, b_e1, W_e2, b_e2, W_x1, b_x1, W_x2,
           W_h1, b_h1, W_h2, b_h2):
    af = atom_feats.astype(jnp.int32)
    row = edge_index[0]
    col = edge_index[1]
    W1ab = W_e1[:2 * D]
    w_r = W_e1[2 * D:2 * D + 1]
    W1d = W_e1[2 * D + 1:]
    coord128 = jnp.pad(coord, ((0, 0), (0, CW - 3)))

    h, A2, B2 = pl.pallas_call(
        _node_pre_body,
        grid=(GN,),
        in_specs=[
            pl.BlockSpec((1, 1, BN), lambda i: (i, 0, 0)),
            pl.BlockSpec((BN, CW), lambda i: (i, 0)),
            _full((N_ATOM, D)),
            _full((2 * D, HID)),
        ],
        out_specs=[
            pl.BlockSpec((BN, D), lambda i: (i, 0)),
            pl.BlockSpec((BN, WID), lambda i: (i, 0)),
            pl.BlockSpec((BN, WID), lambda i: (i, 0)),
        ],
        out_shape=[
            jax.ShapeDtypeStruct((N, D), jnp.float32),
            jax.ShapeDtypeStruct((N, WID), jnp.float32),
            jax.ShapeDtypeStruct((N, WID), jnp.float32),
        ],
    )(af.reshape(GN, 1, BN), coord128, atom_table, W1ab)

    T = pl.pallas_call(
        _node_pre_t_body,
        in_specs=[_full((N_EDGE_T, EDGE_D)), _full((EDGE_D, HID))],
        out_specs=_full((N_EDGE_T, HID)),
        out_shape=jax.ShapeDtypeStruct((N_EDGE_T, HID), jnp.float32),
    )(edge_table, W1d)

    ga, gb = _sc_gather(A2, B2, row, col)

    m, t16, edge_attr, rc0, rc1 = pl.pallas_call(
        _edge_body,
        grid=(GE,),
        in_specs=[
            pl.BlockSpec((BE, WID), lambda i: (i, 0)),
            pl.BlockSpec((BE, WID), lambda i: (i, 0)),
            pl.BlockSpec((1, 1, BE), lambda i: (i, 0, 0)),
            pl.BlockSpec((1, 1, BE), lambda i: (i, 0, 0)),
            _full((N_EDGE_T, HID)),
            _full((1, HID)),
            _full((1, HID)),
            _full((HID, HID)),
            _full((1, HID)),
            _full((HID, HID)),
            _full((1, HID)),
            _full((1, HID)),
            _full((N_EDGE_T, EDGE_D)),
        ],
        out_specs=[
            pl.BlockSpec((BE, D), lambda i: (i, 0)),
            pl.BlockSpec((BE, EDGE_D), lambda i: (i, 0)),
            pl.BlockSpec((BE, EDGE_D), lambda i: (i, 0)),
            pl.BlockSpec((BE, L), lambda i: (i, 0)),
            pl.BlockSpec((BE, L), lambda i: (i, 0)),
        ],
        out_shape=[
            jax.ShapeDtypeStruct((E, D), jnp.float32),
            jax.ShapeDtypeStruct((E, EDGE_D), jnp.float32),
            jax.ShapeDtypeStruct((E, EDGE_D), jnp.float32),
            jax.ShapeDtypeStruct((E, L), jnp.int32),
            jax.ShapeDtypeStruct((E, L), jnp.int32),
        ],
    )(ga, gb, edge_type_ids.reshape(GE, 1, BE), row.reshape(GE, 1, BE), T, w_r,
      b_e1.reshape(1, HID), W_e2, b_e2.reshape(1, HID), W_x1,
      b_x1.reshape(1, HID), W_x2.reshape(1, HID), edge_table)

    m2 = (m.reshape(E // 8, 8, NS, L).transpose(2, 0, 1, 3)
          .reshape(NS, E // 8, 128))
    t16p = t16.reshape(E // 8, 128)
    rc = jnp.stack([rc0.reshape(E // 8, 128), rc1.reshape(E // 8, 128)])
    aggm_p, partt = _sc_scatter(m2, t16p, rc)
    aggm = jnp.concatenate(
        [aggm_p[0].reshape(NS, SC_N, L), aggm_p[1].reshape(NS, SC_N, L)],
        axis=1).transpose(1, 0, 2).reshape(N, HID)

    coord16 = coord128[:, :EDGE_D]
    c16p = jnp.stack([
        jnp.pad(coord16[c * SC_N:(c + 1) * SC_N].reshape(SC_N // 8, 128),
                ((0, ACC_R - SC_N // 8), (0, 0)))
        for c in range(NC)])
    h_out, cout_p = pl.pallas_call(
        _node_post_body,
        grid=(GP,),
        in_specs=[
            pl.BlockSpec((BP, D), lambda i: (i, 0)),
            pl.BlockSpec((BP, HID), lambda i: (i, 0)),
            pl.BlockSpec((1, NS, ACC_R, 128), lambda i: (i, 0, 0, 0)),
            pl.BlockSpec((1, ACC_R, 128), lambda i: (i, 0, 0)),
            _full((D + HID, HID)),
            _full((1, HID)),
            _full((HID, D)),
            _full((1, D)),
        ],
        out_specs=[
            pl.BlockSpec((BP, D), lambda i: (i, 0)),
            pl.BlockSpec((1, ACC_R, 128), lambda i: (i, 0, 0)),
        ],
        out_shape=[
            jax.ShapeDtypeStruct((N, D), jnp.float32),
            jax.ShapeDtypeStruct((NC, ACC_R, 128), jnp.float32),
        ],
    )(h, aggm, partt, c16p, W_h1, b_h1.reshape(1, HID),
      W_h2, b_h2.reshape(1, D))

    cout = jnp.concatenate(
        [cout_p[c].reshape(ACC_N, EDGE_D)[:SC_N] for c in range(NC)])
    return h_out, cout[:, :3], edge_attr


# speedup vs baseline: 1.2629x; 1.2629x over previous
import functools

import jax
import jax.numpy as jnp
from jax import lax
from jax.experimental import pallas as pl
from jax.experimental.pallas import tpu as pltpu
from jax.experimental.pallas import tpu_sc as plsc

N = 10000
E = 160000
D = 256
EDGE_D = 16
N_ATOM = 100
N_EDGE_T = 16
HID = 256
CW = 128
WID = D + CW

NC = 2
NS = 16
NW = NC * NS
L = 16

ECH = 128
NCHUNK = E // ECH
G_ITERS = -(-NCHUNK // NW)
SR1 = 160
NCH1 = E // (SR1 * 8)
SR2 = 80
NCH2 = E // (SR2 * 8)
S2_ITERS = -(-NCH2 // NS)

SC_N = N // NC
ACC_N = 5120
TRASH = 5100
ACC_R = ACC_N // 8

BN = 2000
GN = N // BN
BP = SC_N
GP = NC
BE = 3200
GE = E // BE


def _silu(x):
    return x * jax.nn.sigmoid(x)


def _node_pre_body(af_ref, c128_ref, atab_ref, w1ab_ref,
                   h_ref, a_ref, b_ref):
    ids = af_ref[0, 0, :]
    oh = (ids[:, None] == lax.broadcasted_iota(jnp.int32, (BN, N_ATOM), 1)
          ).astype(jnp.float32)
    h = jnp.dot(oh, atab_ref[...], preferred_element_type=jnp.float32)
    h_ref[...] = h
    c = c128_ref[...]
    a_ref[:, 0:D] = jnp.dot(h, w1ab_ref[0:D, :], preferred_element_type=jnp.float32)
    a_ref[:, D:WID] = c
    b_ref[:, 0:D] = jnp.dot(h, w1ab_ref[D:2 * D, :], preferred_element_type=jnp.float32)
    b_ref[:, D:WID] = -c


def _node_pre_t_body(etab_ref, w1d_ref, t_ref):
    t_ref[...] = jnp.dot(etab_ref[...], w1d_ref[...],
                         preferred_element_type=jnp.float32)


def _edge_body(ga_ref, gb_ref, et_ref, row_ref, t_ref, wr_ref, be1_ref,
               we2_ref, be2_ref, wx1_ref, bx1_ref, wx2_ref, etab_ref,
               m_ref, t16_ref, ea_ref, rc0_ref, rc1_ref):
    cd = ga_ref[:, D:WID] + gb_ref[:, D:WID]
    radial = jnp.sum(cd * cd, axis=1, keepdims=True)
    ids = et_ref[0, 0, :]
    oh = (ids[:, None] == lax.broadcasted_iota(jnp.int32, (BE, N_EDGE_T), 1)
          ).astype(jnp.float32)
    pre = (ga_ref[:, 0:D] + gb_ref[:, 0:D] + radial * wr_ref[...]
           + jnp.dot(oh, t_ref[...], preferred_element_type=jnp.float32)
           + be1_ref[...])
    m1 = _silu(pre)
    m = _silu(jnp.dot(m1, we2_ref[...], preferred_element_type=jnp.float32)
              + be2_ref[...])
    m_ref[...] = m
    m2 = _silu(jnp.dot(m, wx1_ref[...], preferred_element_type=jnp.float32)
               + bx1_ref[...])
    g = jnp.sum(m2 * wx2_ref[...], axis=1, keepdims=True)
    lane = lax.broadcasted_iota(jnp.int32, (BE, EDGE_D), 1)
    t16_ref[...] = jnp.where(lane == 3, 1.0, cd[:, 0:EDGE_D] * g)
    ea_ref[...] = jnp.dot(oh, etab_ref[...], preferred_element_type=jnp.float32)
    r = row_ref[0, 0, :][:, None]
    rc0_ref[...] = jnp.broadcast_to(jnp.where(r < SC_N, r, TRASH), (BE, L))
    rc1_ref[...] = jnp.broadcast_to(jnp.where(r >= SC_N, r - SC_N, TRASH),
                                    (BE, L))


def _node_post_body(h_ref, aggm_ref, pt_ref, c16p_ref,
                    wh1_ref, bh1_ref, wh2_ref, bh2_ref,
                    hout_ref, cout_ref):
    h = h_ref[...]
    u = _silu(jnp.dot(h, wh1_ref[0:D, :], preferred_element_type=jnp.float32)
              + jnp.dot(aggm_ref[...], wh1_ref[D:D + HID, :],
                        preferred_element_type=jnp.float32)
              + bh1_ref[...])
    hout_ref[...] = h + jnp.dot(u, wh2_ref[...],
                                preferred_element_type=jnp.float32) + bh2_ref[...]
    acp = jnp.sum(pt_ref[...], axis=(0, 1))
    rr = lax.broadcasted_iota(jnp.int32, (128, 128), 0)
    cc = lax.broadcasted_iota(jnp.int32, (128, 128), 1)
    msel = (((rr // L) == (cc // L)) & (rr % L == 3)).astype(jnp.float32)
    deg = jnp.dot(acp, msel, preferred_element_type=jnp.float32)
    cout_ref[0] = c16p_ref[0] + acp / jnp.maximum(deg, 1.0)


_sc_mesh = plsc.VectorSubcoreMesh(core_axis_name="c", subcore_axis_name="s")


@functools.partial(
    pl.kernel,
    out_type=(jax.ShapeDtypeStruct((E, WID), jnp.float32),
              jax.ShapeDtypeStruct((E, WID), jnp.float32)),
    scratch_types=[
        pltpu.VMEM((ECH,), jnp.int32),
        pltpu.VMEM((ECH,), jnp.int32),
        pltpu.VMEM((ECH, WID), jnp.float32),
        pltpu.VMEM((ECH, WID), jnp.float32),
        pltpu.SemaphoreType.DMA,
        pltpu.SemaphoreType.DMA,
    ],
    mesh=_sc_mesh,
)
def _sc_gather(a_hbm, b_hbm, row_hbm, col_hbm,
               ga_hbm, gb_hbm, idx1_v, idx2_v, gbuf, gbuf2, sem, sem2):
    c = lax.axis_index("c")
    s = lax.axis_index("s")
    w = s * NC + c

    def body(i, carry):
        ch = w + i * NW

        @pl.when(ch < NCHUNK)
        def _():
            base = ch * ECH
            pltpu.sync_copy(row_hbm.at[pl.ds(base, ECH)], idx1_v)
            pltpu.sync_copy(col_hbm.at[pl.ds(base, ECH)], idx2_v)
            cp1 = pltpu.async_copy(a_hbm.at[idx1_v], gbuf, sem)
            cp2 = pltpu.async_copy(b_hbm.at[idx2_v], gbuf2, sem2)
            cp1.wait()
            cp2.wait()
            pltpu.sync_copy(gbuf, ga_hbm.at[pl.ds(base, ECH), :])
            pltpu.sync_copy(gbuf2, gb_hbm.at[pl.ds(base, ECH), :])

        return carry

    lax.fori_loop(0, G_ITERS, body, 0)


@functools.partial(
    pl.kernel,
    out_type=(jax.ShapeDtypeStruct((NC, NS, SC_N // 8, 128), jnp.float32),
              jax.ShapeDtypeStruct((NC, NS, ACC_R, 128), jnp.float32)),
    scratch_types=[
        pltpu.VMEM((ACC_R, 128), jnp.float32),
        pltpu.VMEM((SR1, 128), jnp.float32),
        pltpu.VMEM((SR1, 128), jnp.int32),
        pltpu.SemaphoreType.DMA,
        pltpu.SemaphoreType.DMA,
    ],
    mesh=_sc_mesh,
    compiler_params=pltpu.CompilerParams(needs_layout_passes=False),
)
def _sc_scatter(m_hbm, t16_hbm, rc_hbm, aggm_hbm, partt_hbm,
                acc, vbuf, rbuf, sem, sem2):
    c = lax.axis_index("c")
    s = lax.axis_index("s")
    iota = lax.broadcasted_iota(jnp.int32, (L,), 0)
    zero = jnp.zeros((L,), jnp.float32)

    def zr(r, carry):
        for k in range(128 // L):
            acc[r, pl.ds(k * L, L)] = zero
        return carry

    def scatter_rows(j):
        r_, lo = j >> 3, (j & 7) * L
        rb = rbuf[r_, pl.ds(lo, L)]
        rowv = lax.shift_right_logical(rb, 3)
        colv = lax.shift_left(jnp.bitwise_and(rb, 7), 4) + iota
        plsc.addupdate_scatter(acc, [rowv, colv], vbuf[r_, pl.ds(lo, L)])

    lax.fori_loop(0, ACC_R, zr, 0)

    def body(i, carry):
        base = i * SR1
        cp1 = pltpu.async_copy(m_hbm.at[s, pl.ds(base, SR1), :], vbuf, sem)
        cp2 = pltpu.async_copy(rc_hbm.at[c, pl.ds(base, SR1), :], rbuf, sem2)
        cp1.wait()
        cp2.wait()
        for j in range(SR1 * 8):
            scatter_rows(j)
        return carry

    lax.fori_loop(0, NCH1, body, 0)
    pltpu.sync_copy(acc.at[pl.ds(0, SC_N // 8), :], aggm_hbm.at[c, s])

    lax.fori_loop(0, ACC_R, zr, 0)

    def body2(i, carry):
        ch = s + i * NS

        @pl.when(ch < NCH2)
        def _():
            base = ch * SR2
            cp1 = pltpu.async_copy(t16_hbm.at[pl.ds(base, SR2), :],
                                   vbuf.at[pl.ds(0, SR2), :], sem)
            cp2 = pltpu.async_copy(rc_hbm.at[c, pl.ds(base, SR2), :],
                                   rbuf.at[pl.ds(0, SR2), :], sem2)
            cp1.wait()
            cp2.wait()
            for j in range(SR2 * 8):
                scatter_rows(j)

        return carry

    lax.fori_loop(0, S2_ITERS, body2, 0)
    pltpu.sync_copy(acc, partt_hbm.at[c, s])


def _full(x):
    return pl.BlockSpec(x, lambda *_: tuple(0 for _ in x))


def kernel(atom_feats, coord, edge_index, edge_type_ids, atom_table,
           edge_table, W_e1, b_e1, W_e2, b_e2, W_x1, b_x1, W_x2,
           W_h1, b_h1, W_h2, b_h2):
    af = atom_feats.astype(jnp.int32)
    row = edge_index[0]
    col = edge_index[1]
    W1ab = W_e1[:2 * D]
    w_r = W_e1[2 * D:2 * D + 1]
    W1d = W_e1[2 * D + 1:]
    coord128 = jnp.pad(coord, ((0, 0), (0, CW - 3)))

    h, A2, B2 = pl.pallas_call(
        _node_pre_body,
        grid=(GN,),
        in_specs=[
            pl.BlockSpec((1, 1, BN), lambda i: (i, 0, 0)),
            pl.BlockSpec((BN, CW), lambda i: (i, 0)),
            _full((N_ATOM, D)),
            _full((2 * D, HID)),
        ],
        out_specs=[
            pl.BlockSpec((BN, D), lambda i: (i, 0)),
            pl.BlockSpec((BN, WID), lambda i: (i, 0)),
            pl.BlockSpec((BN, WID), lambda i: (i, 0)),
        ],
        out_shape=[
            jax.ShapeDtypeStruct((N, D), jnp.float32),
            jax.ShapeDtypeStruct((N, WID), jnp.float32),
            jax.ShapeDtypeStruct((N, WID), jnp.float32),
        ],
    )(af.reshape(GN, 1, BN), coord128, atom_table, W1ab)

    T = pl.pallas_call(
        _node_pre_t_body,
        in_specs=[_full((N_EDGE_T, EDGE_D)), _full((EDGE_D, HID))],
        out_specs=_full((N_EDGE_T, HID)),
        out_shape=jax.ShapeDtypeStruct((N_EDGE_T, HID), jnp.float32),
    )(edge_table, W1d)

    ga, gb = _sc_gather(A2, B2, row, col)

    m, t16, edge_attr, rc0, rc1 = pl.pallas_call(
        _edge_body,
        grid=(GE,),
        in_specs=[
            pl.BlockSpec((BE, WID), lambda i: (i, 0)),
            pl.BlockSpec((BE, WID), lambda i: (i, 0)),
            pl.BlockSpec((1, 1, BE), lambda i: (i, 0, 0)),
            pl.BlockSpec((1, 1, BE), lambda i: (i, 0, 0)),
            _full((N_EDGE_T, HID)),
            _full((1, HID)),
            _full((1, HID)),
            _full((HID, HID)),
            _full((1, HID)),
            _full((HID, HID)),
            _full((1, HID)),
            _full((1, HID)),
            _full((N_EDGE_T, EDGE_D)),
        ],
        out_specs=[
            pl.BlockSpec((BE, D), lambda i: (i, 0)),
            pl.BlockSpec((BE, EDGE_D), lambda i: (i, 0)),
            pl.BlockSpec((BE, EDGE_D), lambda i: (i, 0)),
            pl.BlockSpec((BE, L), lambda i: (i, 0)),
            pl.BlockSpec((BE, L), lambda i: (i, 0)),
        ],
        out_shape=[
            jax.ShapeDtypeStruct((E, D), jnp.float32),
            jax.ShapeDtypeStruct((E, EDGE_D), jnp.float32),
            jax.ShapeDtypeStruct((E, EDGE_D), jnp.float32),
            jax.ShapeDtypeStruct((E, L), jnp.int32),
            jax.ShapeDtypeStruct((E, L), jnp.int32),
        ],
    )(ga, gb, edge_type_ids.reshape(GE, 1, BE), row.reshape(GE, 1, BE), T, w_r,
      b_e1.reshape(1, HID), W_e2, b_e2.reshape(1, HID), W_x1,
      b_x1.reshape(1, HID), W_x2.reshape(1, HID), edge_table)

    m2 = (m.reshape(E // 8, 8, NS, L).transpose(2, 0, 1, 3)
          .reshape(NS, E // 8, 128))
    t16p = t16.reshape(E // 8, 128)
    rc = jnp.stack([rc0.reshape(E // 8, 128), rc1.reshape(E // 8, 128)])
    aggm_p, partt = _sc_scatter(m2, t16p, rc)
    aggm = jnp.concatenate(
        [aggm_p[0].reshape(NS, SC_N, L), aggm_p[1].reshape(NS, SC_N, L)],
        axis=1).transpose(1, 0, 2).reshape(N, HID)

    coord16 = coord128[:, :EDGE_D]
    c16p = jnp.stack([
        jnp.pad(coord16[c * SC_N:(c + 1) * SC_N].reshape(SC_N // 8, 128),
                ((0, ACC_R - SC_N // 8), (0, 0)))
        for c in range(NC)])
    h_out, cout_p = pl.pallas_call(
        _node_post_body,
        grid=(GP,),
        in_specs=[
            pl.BlockSpec((BP, D), lambda i: (i, 0)),
            pl.BlockSpec((BP, HID), lambda i: (i, 0)),
            pl.BlockSpec((1, NS, ACC_R, 128), lambda i: (i, 0, 0, 0)),
            pl.BlockSpec((1, ACC_R, 128), lambda i: (i, 0, 0)),
            _full((D + HID, HID)),
            _full((1, HID)),
            _full((HID, D)),
            _full((1, D)),
        ],
        out_specs=[
            pl.BlockSpec((BP, D), lambda i: (i, 0)),
            pl.BlockSpec((1, ACC_R, 128), lambda i: (i, 0, 0)),
        ],
        out_shape=[
            jax.ShapeDtypeStruct((N, D), jnp.float32),
            jax.ShapeDtypeStruct((NC, ACC_R, 128), jnp.float32),
        ],
    )(h, aggm, partt, c16p, W_h1, b_h1.reshape(1, HID),
      W_h2, b_h2.reshape(1, D))

    cout = jnp.concatenate(
        [cout_p[c].reshape(ACC_N, EDGE_D)[:SC_N] for c in range(NC)])
    return h_out, cout[:, :3], edge_attr


# speedup vs baseline: 1.2658x; 1.0023x over previous
import functools

import jax
import jax.numpy as jnp
from jax import lax
from jax.experimental import pallas as pl
from jax.experimental.pallas import tpu as pltpu
from jax.experimental.pallas import tpu_sc as plsc

N = 10000
E = 160000
D = 256
EDGE_D = 16
N_ATOM = 100
N_EDGE_T = 16
HID = 256
CW = 128
WID = D + CW

NC = 2
NS = 16
NW = NC * NS
L = 16

ECH = 128
NCHUNK = E // ECH
G_ITERS = -(-NCHUNK // NW)
SR1 = 160
NCH1 = E // (SR1 * 8)
SR2 = 80
NCH2 = E // (SR2 * 8)
S2_ITERS = -(-NCH2 // NS)

SC_N = N // NC
ACC_N = 5120
TRASH = 5100
ACC_R = ACC_N // 8

BN = 2000
GN = N // BN
BP = SC_N
GP = NC
BE = 3200
GE = E // BE


def _silu(x):
    return x * jax.nn.sigmoid(x)


def _node_pre_body(af_ref, c128_ref, atab_ref, w1ab_ref,
                   h_ref, a_ref, b_ref):
    ids = af_ref[0, 0, :]
    oh = (ids[:, None] == lax.broadcasted_iota(jnp.int32, (BN, N_ATOM), 1)
          ).astype(jnp.float32)
    h = jnp.dot(oh, atab_ref[...], preferred_element_type=jnp.float32)
    h_ref[...] = h
    c = c128_ref[...]
    a_ref[:, 0:D] = jnp.dot(h, w1ab_ref[0:D, :], preferred_element_type=jnp.float32)
    a_ref[:, D:WID] = c
    b_ref[:, 0:D] = jnp.dot(h, w1ab_ref[D:2 * D, :], preferred_element_type=jnp.float32)
    b_ref[:, D:WID] = -c


def _node_pre_t_body(etab_ref, w1d_ref, t_ref):
    t_ref[...] = jnp.dot(etab_ref[...], w1d_ref[...],
                         preferred_element_type=jnp.float32)


def _edge_body(ga_ref, gb_ref, et_ref, row_ref, t_ref, wr_ref, be1_ref,
               we2_ref, be2_ref, wx1_ref, bx1_ref, wx2_ref, etab_ref,
               m_ref, t16_ref, ea_ref, rc0_ref, rc1_ref):
    cd = ga_ref[:, D:WID] + gb_ref[:, D:WID]
    radial = jnp.sum(cd * cd, axis=1, keepdims=True)
    ids = et_ref[0, 0, :]
    oh = (ids[:, None] == lax.broadcasted_iota(jnp.int32, (BE, N_EDGE_T), 1)
          ).astype(jnp.float32)
    pre = (ga_ref[:, 0:D] + gb_ref[:, 0:D] + radial * wr_ref[...]
           + jnp.dot(oh, t_ref[...], preferred_element_type=jnp.float32)
           + be1_ref[...])
    m1 = _silu(pre)
    m = _silu(jnp.dot(m1, we2_ref[...], preferred_element_type=jnp.float32)
              + be2_ref[...])
    m_ref[...] = m
    m2 = _silu(jnp.dot(m, wx1_ref[...], preferred_element_type=jnp.float32)
               + bx1_ref[...])
    g = jnp.sum(m2 * wx2_ref[...], axis=1, keepdims=True)
    lane = lax.broadcasted_iota(jnp.int32, (BE, EDGE_D), 1)
    t16_ref[...] = jnp.where(lane == 3, 1.0, cd[:, 0:EDGE_D] * g)
    ea_ref[...] = jnp.dot(oh, etab_ref[...], preferred_element_type=jnp.float32)
    r = row_ref[0, 0, :][:, None]
    rc0_ref[...] = jnp.broadcast_to(jnp.where(r < SC_N, r, TRASH), (BE, L))
    rc1_ref[...] = jnp.broadcast_to(jnp.where(r >= SC_N, r - SC_N, TRASH),
                                    (BE, L))


def _node_post_body(h_ref, aggm_ref, pt_ref, c16p_ref,
                    wh1_ref, bh1_ref, wh2_ref, bh2_ref,
                    hout_ref, cout_ref):
    h = h_ref[...]
    u = _silu(jnp.dot(h, wh1_ref[0:D, :], preferred_element_type=jnp.float32)
              + jnp.dot(aggm_ref[...], wh1_ref[D:D + HID, :],
                        preferred_element_type=jnp.float32)
              + bh1_ref[...])
    hout_ref[...] = h + jnp.dot(u, wh2_ref[...],
                                preferred_element_type=jnp.float32) + bh2_ref[...]
    acp = jnp.sum(pt_ref[...], axis=(0, 1))
    rr = lax.broadcasted_iota(jnp.int32, (128, 128), 0)
    cc = lax.broadcasted_iota(jnp.int32, (128, 128), 1)
    msel = (((rr // L) == (cc // L)) & (rr % L == 3)).astype(jnp.float32)
    deg = jnp.dot(acp, msel, preferred_element_type=jnp.float32)
    cout_ref[0] = c16p_ref[0] + acp / jnp.maximum(deg, 1.0)


_sc_mesh = plsc.VectorSubcoreMesh(core_axis_name="c", subcore_axis_name="s")


@functools.partial(
    pl.kernel,
    out_type=(jax.ShapeDtypeStruct((E, WID), jnp.float32),
              jax.ShapeDtypeStruct((E, WID), jnp.float32)),
    scratch_types=[
        pltpu.VMEM((ECH,), jnp.int32),
        pltpu.VMEM((ECH,), jnp.int32),
        pltpu.VMEM((ECH, WID), jnp.float32),
        pltpu.VMEM((ECH, WID), jnp.float32),
        pltpu.SemaphoreType.DMA,
        pltpu.SemaphoreType.DMA,
        pltpu.SemaphoreType.DMA,
        pltpu.SemaphoreType.DMA,
    ],
    mesh=_sc_mesh,
)
def _sc_gather(a_hbm, b_hbm, row_hbm, col_hbm,
               ga_hbm, gb_hbm, idx1_v, idx2_v, gbuf, gbuf2,
               sem, sem2, sem3, sem4):
    c = lax.axis_index("c")
    s = lax.axis_index("s")
    w = s * NC + c

    def body(i, carry):
        ch = w + i * NW

        @pl.when(ch < NCHUNK)
        def _():
            base = ch * ECH
            ci1 = pltpu.async_copy(row_hbm.at[pl.ds(base, ECH)], idx1_v, sem3)
            ci2 = pltpu.async_copy(col_hbm.at[pl.ds(base, ECH)], idx2_v, sem4)
            ci1.wait()
            ci2.wait()
            cp1 = pltpu.async_copy(a_hbm.at[idx1_v], gbuf, sem)
            cp2 = pltpu.async_copy(b_hbm.at[idx2_v], gbuf2, sem2)
            cp1.wait()
            cp2.wait()
            cw1 = pltpu.async_copy(gbuf, ga_hbm.at[pl.ds(base, ECH), :], sem3)
            cw2 = pltpu.async_copy(gbuf2, gb_hbm.at[pl.ds(base, ECH), :], sem4)
            cw1.wait()
            cw2.wait()

        return carry

    lax.fori_loop(0, G_ITERS, body, 0)


@functools.partial(
    pl.kernel,
    out_type=(jax.ShapeDtypeStruct((NC, NS, SC_N // 8, 128), jnp.float32),
              jax.ShapeDtypeStruct((NC, NS, ACC_R, 128), jnp.float32)),
    scratch_types=[
        pltpu.VMEM((ACC_R, 128), jnp.float32),
        pltpu.VMEM((SR1, 128), jnp.float32),
        pltpu.VMEM((SR1, 128), jnp.int32),
        pltpu.SemaphoreType.DMA,
        pltpu.SemaphoreType.DMA,
    ],
    mesh=_sc_mesh,
    compiler_params=pltpu.CompilerParams(needs_layout_passes=False),
)
def _sc_scatter(m_hbm, t16_hbm, rc_hbm, aggm_hbm, partt_hbm,
                acc, vbuf, rbuf, sem, sem2):
    c = lax.axis_index("c")
    s = lax.axis_index("s")
    iota = lax.broadcasted_iota(jnp.int32, (L,), 0)
    zero = jnp.zeros((L,), jnp.float32)

    def zr(r, carry):
        for k in range(128 // L):
            acc[r, pl.ds(k * L, L)] = zero
        return carry

    def scatter_rows(j):
        r_, lo = j >> 3, (j & 7) * L
        rb = rbuf[r_, pl.ds(lo, L)]
        rowv = lax.shift_right_logical(rb, 3)
        colv = lax.shift_left(jnp.bitwise_and(rb, 7), 4) + iota
        plsc.addupdate_scatter(acc, [rowv, colv], vbuf[r_, pl.ds(lo, L)])

    lax.fori_loop(0, ACC_R, zr, 0)

    def body(i, carry):
        base = i * SR1
        cp1 = pltpu.async_copy(m_hbm.at[s, pl.ds(base, SR1), :], vbuf, sem)
        cp2 = pltpu.async_copy(rc_hbm.at[c, pl.ds(base, SR1), :], rbuf, sem2)
        cp1.wait()
        cp2.wait()
        for j in range(SR1 * 8):
            scatter_rows(j)
        return carry

    lax.fori_loop(0, NCH1, body, 0)
    pltpu.sync_copy(acc.at[pl.ds(0, SC_N // 8), :], aggm_hbm.at[c, s])

    lax.fori_loop(0, ACC_R, zr, 0)

    def body2(i, carry):
        ch = s + i * NS

        @pl.when(ch < NCH2)
        def _():
            base = ch * SR2
            cp1 = pltpu.async_copy(t16_hbm.at[pl.ds(base, SR2), :],
                                   vbuf.at[pl.ds(0, SR2), :], sem)
            cp2 = pltpu.async_copy(rc_hbm.at[c, pl.ds(base, SR2), :],
                                   rbuf.at[pl.ds(0, SR2), :], sem2)
            cp1.wait()
            cp2.wait()
            for j in range(SR2 * 8):
                scatter_rows(j)

        return carry

    lax.fori_loop(0, S2_ITERS, body2, 0)
    pltpu.sync_copy(acc, partt_hbm.at[c, s])


def _full(x):
    return pl.BlockSpec(x, lambda *_: tuple(0 for _ in x))


def kernel(atom_feats, coord, edge_index, edge_type_ids, atom_table,
           edge_table, W_e1, b_e1, W_e2, b_e2, W_x1, b_x1, W_x2,
           W_h1, b_h1, W_h2, b_h2):
    af = atom_feats.astype(jnp.int32)
    row = edge_index[0]
    col = edge_index[1]
    W1ab = W_e1[:2 * D]
    w_r = W_e1[2 * D:2 * D + 1]
    W1d = W_e1[2 * D + 1:]
    coord128 = jnp.pad(coord, ((0, 0), (0, CW - 3)))

    h, A2, B2 = pl.pallas_call(
        _node_pre_body,
        grid=(GN,),
        in_specs=[
            pl.BlockSpec((1, 1, BN), lambda i: (i, 0, 0)),
            pl.BlockSpec((BN, CW), lambda i: (i, 0)),
            _full((N_ATOM, D)),
            _full((2 * D, HID)),
        ],
        out_specs=[
            pl.BlockSpec((BN, D), lambda i: (i, 0)),
            pl.BlockSpec((BN, WID), lambda i: (i, 0)),
            pl.BlockSpec((BN, WID), lambda i: (i, 0)),
        ],
        out_shape=[
            jax.ShapeDtypeStruct((N, D), jnp.float32),
            jax.ShapeDtypeStruct((N, WID), jnp.float32),
            jax.ShapeDtypeStruct((N, WID), jnp.float32),
        ],
    )(af.reshape(GN, 1, BN), coord128, atom_table, W1ab)

    T = pl.pallas_call(
        _node_pre_t_body,
        in_specs=[_full((N_EDGE_T, EDGE_D)), _full((EDGE_D, HID))],
        out_specs=_full((N_EDGE_T, HID)),
        out_shape=jax.ShapeDtypeStruct((N_EDGE_T, HID), jnp.float32),
    )(edge_table, W1d)

    ga, gb = _sc_gather(A2, B2, row, col)

    m, t16, edge_attr, rc0, rc1 = pl.pallas_call(
        _edge_body,
        grid=(GE,),
        in_specs=[
            pl.BlockSpec((BE, WID), lambda i: (i, 0)),
            pl.BlockSpec((BE, WID), lambda i: (i, 0)),
            pl.BlockSpec((1, 1, BE), lambda i: (i, 0, 0)),
            pl.BlockSpec((1, 1, BE), lambda i: (i, 0, 0)),
            _full((N_EDGE_T, HID)),
            _full((1, HID)),
            _full((1, HID)),
            _full((HID, HID)),
            _full((1, HID)),
            _full((HID, HID)),
            _full((1, HID)),
            _full((1, HID)),
            _full((N_EDGE_T, EDGE_D)),
        ],
        out_specs=[
            pl.BlockSpec((BE, D), lambda i: (i, 0)),
            pl.BlockSpec((BE, EDGE_D), lambda i: (i, 0)),
            pl.BlockSpec((BE, EDGE_D), lambda i: (i, 0)),
            pl.BlockSpec((BE, L), lambda i: (i, 0)),
            pl.BlockSpec((BE, L), lambda i: (i, 0)),
        ],
        out_shape=[
            jax.ShapeDtypeStruct((E, D), jnp.float32),
            jax.ShapeDtypeStruct((E, EDGE_D), jnp.float32),
            jax.ShapeDtypeStruct((E, EDGE_D), jnp.float32),
            jax.ShapeDtypeStruct((E, L), jnp.int32),
            jax.ShapeDtypeStruct((E, L), jnp.int32),
        ],
    )(ga, gb, edge_type_ids.reshape(GE, 1, BE), row.reshape(GE, 1, BE), T, w_r,
      b_e1.reshape(1, HID), W_e2, b_e2.reshape(1, HID), W_x1,
      b_x1.reshape(1, HID), W_x2.reshape(1, HID), edge_table)

    m2 = (m.reshape(E // 8, 8, NS, L).transpose(2, 0, 1, 3)
          .reshape(NS, E // 8, 128))
    t16p = t16.reshape(E // 8, 128)
    rc = jnp.stack([rc0.reshape(E // 8, 128), rc1.reshape(E // 8, 128)])
    aggm_p, partt = _sc_scatter(m2, t16p, rc)
    aggm = jnp.concatenate(
        [aggm_p[0].reshape(NS, SC_N, L), aggm_p[1].reshape(NS, SC_N, L)],
        axis=1).transpose(1, 0, 2).reshape(N, HID)

    coord16 = coord128[:, :EDGE_D]
    c16p = jnp.stack([
        jnp.pad(coord16[c * SC_N:(c + 1) * SC_N].reshape(SC_N // 8, 128),
                ((0, ACC_R - SC_N // 8), (0, 0)))
        for c in range(NC)])
    h_out, cout_p = pl.pallas_call(
        _node_post_body,
        grid=(GP,),
        in_specs=[
            pl.BlockSpec((BP, D), lambda i: (i, 0)),
            pl.BlockSpec((BP, HID), lambda i: (i, 0)),
            pl.BlockSpec((1, NS, ACC_R, 128), lambda i: (i, 0, 0, 0)),
            pl.BlockSpec((1, ACC_R, 128), lambda i: (i, 0, 0)),
            _full((D + HID, HID)),
            _full((1, HID)),
            _full((HID, D)),
            _full((1, D)),
        ],
        out_specs=[
            pl.BlockSpec((BP, D), lambda i: (i, 0)),
            pl.BlockSpec((1, ACC_R, 128), lambda i: (i, 0, 0)),
        ],
        out_shape=[
            jax.ShapeDtypeStruct((N, D), jnp.float32),
            jax.ShapeDtypeStruct((NC, ACC_R, 128), jnp.float32),
        ],
    )(h, aggm, partt, c16p, W_h1, b_h1.reshape(1, HID),
      W_h2, b_h2.reshape(1, D))

    cout = jnp.concatenate(
        [cout_p[c].reshape(ACC_N, EDGE_D)[:SC_N] for c in range(NC)])
    return h_out, cout[:, :3], edge_attr
